# baseline (speedup 1.0000x reference)
import jax
import jax.numpy as jnp
from jax.experimental import pallas as pl

_B, _NC, _NF = 8, 1024, 4096
_CC, _CS, _OUT = 512, 256, 512
_T = 512
_G = 32
_EPS_GN = 1e-5


def _knn_interp_body(xf_ref, xct_ref, fc_ref, out_ref):
    xf = xf_ref[0]
    xct = xct_ref[0]
    fc = fc_ref[0]
    sqf = jnp.sum(xf * xf, axis=1, keepdims=True)
    sqc = jnp.sum(xct * xct, axis=0, keepdims=True)
    cross = jax.lax.dot_general(xf, xct, (((1,), (0,)), ((), ())),
                                preferred_element_type=jnp.float32)
    d = jnp.sqrt(jnp.maximum(sqf + sqc - 2.0 * cross, 0.0))

    idx = jax.lax.broadcasted_iota(jnp.int32, d.shape, 1)
    inf = jnp.float32(jnp.inf)
    nbig = jnp.int32(_NC)

    m1 = jnp.min(d, axis=1, keepdims=True)
    i1 = jnp.min(jnp.where(d == m1, idx, nbig), axis=1, keepdims=True)
    dm = jnp.where(idx == i1, inf, d)
    m2 = jnp.min(dm, axis=1, keepdims=True)
    i2 = jnp.min(jnp.where(dm == m2, idx, nbig), axis=1, keepdims=True)
    dm = jnp.where(idx == i2, inf, dm)
    m3 = jnp.min(dm, axis=1, keepdims=True)
    i3 = jnp.min(jnp.where(dm == m3, idx, nbig), axis=1, keepdims=True)

    w1 = 1.0 / (m1 + 1e-12)
    w2 = 1.0 / (m2 + 1e-12)
    w3 = 1.0 / (m3 + 1e-12)
    s = w1 + w2 + w3
    w1, w2, w3 = w1 / s, w2 / s, w3 / s
    zero = m1 <= 1e-12
    w1 = jnp.where(zero, 1.0, w1)
    w2 = jnp.where(zero, 0.0, w2)
    w3 = jnp.where(zero, 0.0, w3)

    a = (jnp.where(idx == i1, w1, 0.0)
         + jnp.where(idx == i2, w2, 0.0)
         + jnp.where(idx == i3, w3, 0.0))
    out_ref[0] = jax.lax.dot_general(a.astype(jnp.bfloat16),
                                     fc.astype(jnp.bfloat16),
                                     (((1,), (0,)), ((), ())),
                                     preferred_element_type=jnp.float32)


def _group_norm_t(h, gamma, beta):
    c = h.shape[1]
    per = c // _G
    gid_r = jax.lax.broadcasted_iota(jnp.int32, (c, c), 0) // per
    gid_c = jax.lax.broadcasted_iota(jnp.int32, (c, c), 1) // per
    p = (gid_r == gid_c).astype(jnp.float32)
    denom = jnp.float32(per * h.shape[0])
    s = jnp.sum(h, axis=0, keepdims=True)
    mu = jax.lax.dot_general(s, p, (((1,), (0,)), ((), ())),
                             preferred_element_type=jnp.float32) / denom
    cen = h - mu
    ss = jnp.sum(cen * cen, axis=0, keepdims=True)
    var = jax.lax.dot_general(ss, p, (((1,), (0,)), ((), ())),
                              preferred_element_type=jnp.float32) / denom
    return cen / jnp.sqrt(var + _EPS_GN) * gamma + beta


def _mlp_body(x1_ref, x2_ref, w1a_ref, w1b_ref, b1_ref, g1_ref, be1_ref,
              w2_ref, b2_ref, g2_ref, be2_ref, out_ref):
    x1 = x1_ref[0].astype(jnp.bfloat16)
    x2 = x2_ref[0].astype(jnp.bfloat16)
    h = (jax.lax.dot_general(x1, w1a_ref[...].astype(jnp.bfloat16),
                             (((1,), (0,)), ((), ())),
                             preferred_element_type=jnp.float32)
         + jax.lax.dot_general(x2, w1b_ref[...].astype(jnp.bfloat16),
                               (((1,), (0,)), ((), ())),
                               preferred_element_type=jnp.float32)
         + b1_ref[...])
    h = jnp.maximum(_group_norm_t(h, g1_ref[...], be1_ref[...]), 0.0)
    h = jax.lax.dot_general(h.astype(jnp.bfloat16),
                            w2_ref[...].astype(jnp.bfloat16),
                            (((1,), (0,)), ((), ())),
                            preferred_element_type=jnp.float32) + b2_ref[...]
    out_ref[0] = jnp.maximum(_group_norm_t(h, g2_ref[...], be2_ref[...]), 0.0)


def kernel(xyz_coarse, feat_coarse, xyz_fine, feat_skip, W1, b1, g1, be1, W2, b2, g2, be2):
    B, Nf, _ = xyz_fine.shape
    Nc = xyz_coarse.shape[1]
    Cc = feat_coarse.shape[2]
    Cs = feat_skip.shape[2]
    out_ch = W1.shape[0]

    xct = jnp.swapaxes(xyz_coarse, 1, 2)
    interp = pl.pallas_call(
        _knn_interp_body,
        grid=(B, Nf // _T),
        in_specs=[
            pl.BlockSpec((1, _T, 3), lambda b, n: (b, n, 0)),
            pl.BlockSpec((1, 3, Nc), lambda b, n: (b, 0, 0)),
            pl.BlockSpec((1, Nc, Cc), lambda b, n: (b, 0, 0)),
        ],
        out_specs=pl.BlockSpec((1, _T, Cc), lambda b, n: (b, n, 0)),
        out_shape=jax.ShapeDtypeStruct((B, Nf, Cc), jnp.float32),
    )(xyz_fine, xct, feat_coarse)

    w1a = jnp.swapaxes(W1[:, :Cc], 0, 1)
    w1b = jnp.swapaxes(W1[:, Cc:], 0, 1)
    w2t = jnp.swapaxes(W2, 0, 1)
    full = lambda shp: pl.BlockSpec(shp, lambda b: tuple(0 for _ in shp))
    out = pl.pallas_call(
        _mlp_body,
        grid=(B,),
        in_specs=[
            pl.BlockSpec((1, Nf, Cc), lambda b: (b, 0, 0)),
            pl.BlockSpec((1, Nf, Cs), lambda b: (b, 0, 0)),
            full((Cc, out_ch)),
            full((Cs, out_ch)),
            full((1, out_ch)),
            full((1, out_ch)),
            full((1, out_ch)),
            full((out_ch, out_ch)),
            full((1, out_ch)),
            full((1, out_ch)),
            full((1, out_ch)),
        ],
        out_specs=pl.BlockSpec((1, Nf, out_ch), lambda b: (b, 0, 0)),
        out_shape=jax.ShapeDtypeStruct((B, Nf, out_ch), jnp.float32),
    )(interp, feat_skip, w1a, w1b, b1[None], g1[None], be1[None],
      w2t, b2[None], g2[None], be2[None])
    return out


# speedup vs baseline: 1.4495x; 1.4495x over previous
import jax
import jax.numpy as jnp
from jax.experimental import pallas as pl

_T = 512
_G = 32
_EPS_GN = 1e-5
_BIG = 3e38


def _knn_interp_body(xfa_ref, xca_ref, fc_ref, out_ref):
    xfa = xfa_ref[0]
    xca = xca_ref[0]
    fc = fc_ref[0]
    d2 = jax.lax.dot_general(xfa, xca, (((1,), (0,)), ((), ())),
                             preferred_element_type=jnp.float32)

    m1 = jnp.min(d2, axis=1, keepdims=True)
    eq1 = d2 == m1
    dm1 = jnp.where(eq1, _BIG, d2)
    m2 = jnp.min(dm1, axis=1, keepdims=True)
    eq2 = dm1 == m2
    dm2 = jnp.where(eq2, _BIG, dm1)
    m3 = jnp.min(dm2, axis=1, keepdims=True)
    eq3 = dm2 == m3

    d1 = jnp.sqrt(jnp.maximum(m1, 0.0))
    d2s = jnp.sqrt(jnp.maximum(m2, 0.0))
    d3s = jnp.sqrt(jnp.maximum(m3, 0.0))
    w1 = 1.0 / (d1 + 1e-12)
    w2 = 1.0 / (d2s + 1e-12)
    w3 = 1.0 / (d3s + 1e-12)
    s = w1 + w2 + w3
    w1, w2, w3 = w1 / s, w2 / s, w3 / s
    zero = d1 <= 1e-12
    w1 = jnp.where(zero, 1.0, w1)
    w2 = jnp.where(zero, 0.0, w2)
    w3 = jnp.where(zero, 0.0, w3)

    a = jnp.where(eq1, w1, jnp.where(eq2, w2, jnp.where(eq3, w3, 0.0)))
    out_ref[0] = jax.lax.dot_general(a.astype(jnp.bfloat16),
                                     fc.astype(jnp.bfloat16),
                                     (((1,), (0,)), ((), ())),
                                     preferred_element_type=jnp.float32)


def _gn_relu(h, p_ref, gamma, beta, denom):
    s = jnp.sum(h, axis=0, keepdims=True)
    q = jnp.sum(h * h, axis=0, keepdims=True)
    mu = jax.lax.dot_general(s, p_ref[...], (((1,), (0,)), ((), ())),
                             preferred_element_type=jnp.float32) / denom
    ex2 = jax.lax.dot_general(q, p_ref[...], (((1,), (0,)), ((), ())),
                              preferred_element_type=jnp.float32) / denom
    var = ex2 - mu * mu
    scale = gamma * jax.lax.rsqrt(var + _EPS_GN)
    shift = beta - mu * scale
    return jnp.maximum(h * scale + shift, 0.0)


def _mlp_body(x1_ref, x2_ref, p_ref, w1a_ref, w1b_ref, b1_ref, g1_ref,
              be1_ref, w2_ref, b2_ref, g2_ref, be2_ref, out_ref):
    x1 = x1_ref[0].astype(jnp.bfloat16)
    x2 = x2_ref[0].astype(jnp.bfloat16)
    denom = jnp.float32(x1.shape[0] * (p_ref.shape[0] // _G))
    h = (jax.lax.dot_general(x1, w1a_ref[...], (((1,), (0,)), ((), ())),
                             preferred_element_type=jnp.float32)
         + jax.lax.dot_general(x2, w1b_ref[...], (((1,), (0,)), ((), ())),
                               preferred_element_type=jnp.float32)
         + b1_ref[...])
    h = _gn_relu(h, p_ref, g1_ref[...], be1_ref[...], denom)
    h = jax.lax.dot_general(h.astype(jnp.bfloat16), w2_ref[...],
                            (((1,), (0,)), ((), ())),
                            preferred_element_type=jnp.float32) + b2_ref[...]
    out_ref[0] = _gn_relu(h, p_ref, g2_ref[...], be2_ref[...], denom)


def kernel(xyz_coarse, feat_coarse, xyz_fine, feat_skip, W1, b1, g1, be1, W2, b2, g2, be2):
    B, Nf, _ = xyz_fine.shape
    Nc = xyz_coarse.shape[1]
    Cc = feat_coarse.shape[2]
    Cs = feat_skip.shape[2]
    out_ch = W1.shape[0]

    sqf = jnp.sum(xyz_fine * xyz_fine, axis=2, keepdims=True)
    ones_f = jnp.ones_like(sqf)
    xfa = jnp.concatenate([-2.0 * xyz_fine, sqf, ones_f], axis=2)
    xct = jnp.swapaxes(xyz_coarse, 1, 2)
    sqc = jnp.sum(xct * xct, axis=1, keepdims=True)
    ones_c = jnp.ones_like(sqc)
    xca = jnp.concatenate([xct, ones_c, sqc], axis=1)

    interp = pl.pallas_call(
        _knn_interp_body,
        grid=(B, Nf // _T),
        in_specs=[
            pl.BlockSpec((1, _T, 5), lambda b, n: (b, n, 0)),
            pl.BlockSpec((1, 5, Nc), lambda b, n: (b, 0, 0)),
            pl.BlockSpec((1, Nc, Cc), lambda b, n: (b, 0, 0)),
        ],
        out_specs=pl.BlockSpec((1, _T, Cc), lambda b, n: (b, n, 0)),
        out_shape=jax.ShapeDtypeStruct((B, Nf, Cc), jnp.float32),
    )(xfa, xca, feat_coarse)

    gid = jnp.arange(out_ch, dtype=jnp.int32) // (out_ch // _G)
    p = (gid[:, None] == gid[None, :]).astype(jnp.float32)
    w1a = jnp.swapaxes(W1[:, :Cc], 0, 1).astype(jnp.bfloat16)
    w1b = jnp.swapaxes(W1[:, Cc:], 0, 1).astype(jnp.bfloat16)
    w2t = jnp.swapaxes(W2, 0, 1).astype(jnp.bfloat16)
    full = lambda shp: pl.BlockSpec(shp, lambda b: tuple(0 for _ in shp))
    out = pl.pallas_call(
        _mlp_body,
        grid=(B,),
        in_specs=[
            pl.BlockSpec((1, Nf, Cc), lambda b: (b, 0, 0)),
            pl.BlockSpec((1, Nf, Cs), lambda b: (b, 0, 0)),
            full((out_ch, out_ch)),
            full((Cc, out_ch)),
            full((Cs, out_ch)),
            full((1, out_ch)),
            full((1, out_ch)),
            full((1, out_ch)),
            full((out_ch, out_ch)),
            full((1, out_ch)),
            full((1, out_ch)),
            full((1, out_ch)),
        ],
        out_specs=pl.BlockSpec((1, Nf, out_ch), lambda b: (b, 0, 0)),
        out_shape=jax.ShapeDtypeStruct((B, Nf, out_ch), jnp.float32),
    )(interp, feat_skip, p, w1a, w1b, b1[None], g1[None], be1[None],
      w2t, b2[None], g2[None], be2[None])
    return out
